# baseline (speedup 1.0000x reference)
import jax
import jax.numpy as jnp
from jax import lax
from jax.experimental import pallas as pl
from jax.experimental.pallas import tpu as pltpu
from jax.experimental.pallas import tpu_sc as plsc

B = 16384
K = 32
NC = 2
NS = 16
NW = NC * NS
BPW = B // NW
CH = 128
NCH = BPW // CH
PAD = 17


def _mf_body(du_hbm, di_hbm, ue_hbm, ie_hbm, ub_hbm, ib_hbm, out_hbm,
             idx_u, idx_i, urows, irows, ubv, ibv, spad, outv, sem):
    wid = lax.axis_index("s") * NC + lax.axis_index("c")

    pltpu.sync_copy(du_hbm.at[wid], idx_u)
    pltpu.sync_copy(di_hbm.at[wid], idx_i)

    copies = []
    for c in range(NCH):
        copies.append(pltpu.async_copy(ue_hbm.at[idx_u.at[c]], urows.at[c], sem))
        copies.append(pltpu.async_copy(ie_hbm.at[idx_i.at[c]], irows.at[c], sem))
        copies.append(pltpu.async_copy(ub_hbm.at[idx_u.at[c]], ubv.at[c], sem))
        copies.append(pltpu.async_copy(ib_hbm.at[idx_i.at[c]], ibv.at[c], sem))
    for cp in copies:
        cp.wait()

    lane = jnp.arange(16, dtype=jnp.int32)
    for c in range(NCH):
        @pl.loop(0, CH, unroll=4)
        def _(rr):
            u0 = urows[c, rr, pl.ds(0, 16)]
            u1 = urows[c, rr, pl.ds(16, 16)]
            i0 = irows[c, rr, pl.ds(0, 16)]
            i1 = irows[c, rr, pl.ds(16, 16)]
            t = u0 * i0 + u1 * i1
            plsc.store_scatter(spad, [(c * CH + rr) * PAD + lane], t)

    gpc = CH // 16
    for g in range(BPW // 16):
        base = (g * 16 + lane) * PAD
        acc = (ubv[g // gpc, pl.ds((g % gpc) * 16, 16)] +
               ibv[g // gpc, pl.ds((g % gpc) * 16, 16)])
        for j in range(16):
            acc = acc + plsc.load_gather(spad, [base + j])
        outv[pl.ds(g * 16, 16)] = acc

    pltpu.sync_copy(outv, out_hbm.at[pl.ds(wid * BPW, BPW)])


@jax.jit
def _mf(du, di, u_emb, i_emb, ub, ib):
    mesh = plsc.VectorSubcoreMesh(core_axis_name="c", subcore_axis_name="s")
    return pl.kernel(
        _mf_body,
        out_type=jax.ShapeDtypeStruct((B,), jnp.float32),
        mesh=mesh,
        compiler_params=pltpu.CompilerParams(
            needs_layout_passes=False, use_tc_tiling_on_sc=False),
        scratch_types=[
            pltpu.VMEM((NCH, CH), jnp.int32),
            pltpu.VMEM((NCH, CH), jnp.int32),
            pltpu.VMEM((NCH, CH, K), jnp.float32),
            pltpu.VMEM((NCH, CH, K), jnp.float32),
            pltpu.VMEM((NCH, CH), jnp.float32),
            pltpu.VMEM((NCH, CH), jnp.float32),
            pltpu.VMEM((BPW * PAD,), jnp.float32),
            pltpu.VMEM((BPW,), jnp.float32),
            pltpu.SemaphoreType.DMA,
        ],
    )(du, di, u_emb, i_emb, ub, ib)


def kernel(data_u, data_i, u_emb, i_emb, user_b, item_b):
    du = data_u.astype(jnp.int32).reshape(NW, NCH, CH)
    di = data_i.astype(jnp.int32).reshape(NW, NCH, CH)
    ub = user_b.reshape(-1)
    ib = item_b.reshape(-1)
    return _mf(du, di, u_emb, i_emb, ub, ib)
